# baseline (speedup 1.0000x reference)
import functools

import jax
import jax.numpy as jnp
import numpy as np
from jax import lax
from jax.experimental import pallas as pl
from jax.experimental.pallas import tpu as pltpu
from jax.experimental.pallas import tpu_sc as plsc

_F = 26
_D = 64
_B = 4096
_NPAIR = (_F * (_F - 1)) // 2
_OUT_W = _NPAIR + _F
_NH = 2
_BH = _B // _NH

_PAIRS = [(i, j) for i in range(_F - 1) for j in range(i + 1, _F)]

_NC, _NS = 2, 16
_NW = _NC * _NS
_TOT_H = _BH * _F
_PER_W = _TOT_H // _NW
_CHUNK = 832
_NCHUNK = _PER_W // _CHUNK

_s = np.arange(_TOT_H, dtype=np.int64)
_b, _f = _s % _BH, _s // _BH
_DPERM = (((_b // 8) * (_F // 2) + _f // 2) * 16 + (_b % 8) * 2 + (_f % 2))
_DPERM = _DPERM.astype(np.int32).reshape(_NW, _NCHUNK, _CHUNK)


def _sc_gather_body(xtf_hbm, emb_hbm, lin_hbm, dperm_hbm,
                    out_e, out_l,
                    idxt_v, dp_v, rows_v0, rows_v1, lin_v,
                    gsem, osem):
    wid = lax.axis_index("s") * _NC + lax.axis_index("c")
    base = wid * _PER_W
    pltpu.sync_copy(xtf_hbm.at[pl.ds(base, _PER_W)], idxt_v)
    pltpu.sync_copy(dperm_hbm.at[wid], dp_v)
    lin_g = pltpu.async_copy(lin_hbm.at[idxt_v], lin_v, gsem)
    bufs = (rows_v0, rows_v1)
    outs = []
    for c in range(_NCHUNK):
        buf = bufs[c % 2]
        g = pltpu.async_copy(
            emb_hbm.at[idxt_v.at[pl.ds(c * _CHUNK, _CHUNK)]], buf, gsem
        )
        if c >= 2:
            outs[c - 2].wait()
        g.wait()
        outs.append(pltpu.async_copy(buf, out_e.at[dp_v.at[c]], osem))
    lin_g.wait()
    for o in outs[-2:]:
        o.wait()
    pltpu.sync_copy(lin_v, out_l.at[pl.ds(base, _PER_W)])


@functools.cache
def _make_gather():
    return pl.kernel(
        _sc_gather_body,
        out_type=[
            jax.ShapeDtypeStruct((_TOT_H, _D), jnp.float32),
            jax.ShapeDtypeStruct((_TOT_H,), jnp.float32),
        ],
        mesh=plsc.VectorSubcoreMesh(
            core_axis_name="c", subcore_axis_name="s",
            num_cores=_NC, num_subcores=_NS,
        ),
        scratch_types=[
            pltpu.VMEM((_PER_W,), jnp.int32),
            pltpu.VMEM((_NCHUNK, _CHUNK), jnp.int32),
            pltpu.VMEM((_CHUNK, _D), jnp.float32),
            pltpu.VMEM((_CHUNK, _D), jnp.float32),
            pltpu.VMEM((_PER_W,), jnp.float32),
            pltpu.SemaphoreType.DMA,
            pltpu.SemaphoreType.DMA,
        ],
        compiler_params=pltpu.CompilerParams(use_tc_tiling_on_sc=False),
    )


_BBLK = 256
_NBAND = _BBLK // 8
_NBLK_H = _BH // _BBLK


def _pairs_compute(e_ref, lin_ref, out_ref):
    e4 = e_ref[...]
    pieces = [jnp.transpose(e4[:, tc].reshape(_BBLK, 128))
              for tc in range(_F // 2)]

    def slab(f):
        return pieces[f // 2][(f % 2) * _D:(f % 2) * _D + _D, :]

    for p, (i, j) in enumerate(_PAIRS):
        prod = slab(i) * slab(j)
        out_ref[p, :] = jnp.sum(prod, axis=0)
    out_ref[_NPAIR:_OUT_W, :] = lin_ref[...]


def _tc_pairs_body0(e_ref, lin_ref, out_ref):
    _pairs_compute(e_ref, lin_ref, out_ref)


def _tc_pairs_body1(e_ref, lin_ref, prev_ref, out_ref):
    del prev_ref
    _pairs_compute(e_ref, lin_ref, out_ref)


_e_spec = pl.BlockSpec((_NBAND, _F // 2, 8, 128), lambda b: (b, 0, 0, 0))
_lin_spec = pl.BlockSpec((_F, _BBLK), lambda b: (0, b))

_pairs_call0 = pl.pallas_call(
    _tc_pairs_body0,
    out_shape=jax.ShapeDtypeStruct((_OUT_W, _B), jnp.float32),
    grid=(_NBLK_H,),
    in_specs=[_e_spec, _lin_spec],
    out_specs=pl.BlockSpec((_OUT_W, _BBLK), lambda b: (0, b)),
)

_pairs_call1 = pl.pallas_call(
    _tc_pairs_body1,
    out_shape=jax.ShapeDtypeStruct((_OUT_W, _B), jnp.float32),
    grid=(_NBLK_H,),
    in_specs=[
        _e_spec,
        _lin_spec,
        pl.BlockSpec((_OUT_W, _BBLK), lambda b: (0, b + _NBLK_H)),
    ],
    out_specs=pl.BlockSpec((_OUT_W, _BBLK), lambda b: (0, b + _NBLK_H)),
    input_output_aliases={2: 0},
)


def kernel(x, emb_weight, lin_weight):
    xt = x.T
    lin1d = lin_weight.reshape(-1)
    dperm = jnp.asarray(_DPERM)
    gather = _make_gather()
    halves = []
    for h in range(_NH):
        xh = xt[:, h * _BH:(h + 1) * _BH].reshape(_TOT_H)
        e_g, l_g = gather(xh, emb_weight, lin1d, dperm)
        halves.append((e_g.reshape(_BH // 8, _F // 2, 8, 128),
                       l_g.reshape(_F, _BH)))
    out = _pairs_call0(halves[0][0], halves[0][1])
    out = _pairs_call1(halves[1][0], halves[1][1], out)
    return out.T
